# Initial kernel scaffold; baseline (speedup 1.0000x reference)
import jax
import jax.numpy as jnp
from jax.experimental import pallas as pl


def kernel(seq_len, bias_table):
    raise NotImplementedError("write your pallas kernel here")


# speedup vs baseline: 41.4063x; 41.4063x over previous
import functools

import jax
import jax.numpy as jnp
from jax import lax
from jax.experimental import pallas as pl
from jax.experimental.pallas import tpu as pltpu
from jax.experimental.pallas import tpu_sc as plsc

_S = 2048
_H = 16
_PAD = 4096
_NSHIFT = 8
_K = 8


def _expand_bias(shifted_tables):
    mesh = plsc.VectorSubcoreMesh(core_axis_name="c", subcore_axis_name="s")

    @functools.partial(
        pl.kernel,
        mesh=mesh,
        out_type=jax.ShapeDtypeStruct((_H, _S, _S), jnp.float32),
        scratch_types=[
            pltpu.VMEM((_NSHIFT * _PAD,), jnp.float32),
            pltpu.SemaphoreType.DMA,
        ],
        compiler_params=pltpu.CompilerParams(use_tc_tiling_on_sc=False),
    )
    def body(p_hbm, out_hbm, p_v, sem):
        cid = lax.axis_index("c")
        sid = lax.axis_index("s")
        wid = sid * 2 + cid
        h = wid // 2
        i0 = (wid % 2) * (_S // 2)

        pltpu.sync_copy(p_hbm.at[h], p_v)

        def row_copy(i, kk):
            off = (_S - 1) - i
            q = ((_S - 1) - kk) % _NSHIFT
            base = q * _PAD + (off - q)
            return pltpu.make_async_copy(
                p_v.at[pl.ds(base, _S)],
                out_hbm.at[h, i],
                sem,
            )

        def loop(t, carry):
            ibase = i0 + t * _K
            for kk in range(_K):
                row_copy(ibase + kk, kk).start()
            for kk in range(_K):
                row_copy(ibase + kk, kk).wait()
            return carry

        lax.fori_loop(0, (_S // 2) // _K, loop, 0)

    return body(shifted_tables)


def kernel(seq_len, bias_table):
    del seq_len
    rev = bias_table[::-1, :]
    pad_rows = _PAD + _NSHIFT - 1 - rev.shape[0]
    rev = jnp.concatenate(
        [rev, jnp.zeros((pad_rows, _H), bias_table.dtype)], axis=0)
    shifted = jnp.stack(
        [lax.slice_in_dim(rev, q, q + _PAD, axis=0) for q in range(_NSHIFT)],
        axis=0)
    shifted = jnp.transpose(shifted, (2, 0, 1)).reshape(_H, _NSHIFT * _PAD)
    out = _expand_bias(shifted)
    return out[None]


# speedup vs baseline: 41.7611x; 1.0086x over previous
import functools

import jax
import jax.numpy as jnp
from jax import lax
from jax.experimental import pallas as pl
from jax.experimental.pallas import tpu as pltpu
from jax.experimental.pallas import tpu_sc as plsc

_S = 2048
_H = 16
_PAD = 4096
_NSHIFT = 8
_K = 8


def _expand_bias(shifted_tables):
    mesh = plsc.VectorSubcoreMesh(core_axis_name="c", subcore_axis_name="s")

    @functools.partial(
        pl.kernel,
        mesh=mesh,
        out_type=jax.ShapeDtypeStruct((_H, _S, _S), jnp.float32),
        scratch_types=[
            pltpu.VMEM((_NSHIFT * _PAD,), jnp.float32),
            pltpu.SemaphoreType.DMA,
        ],
        compiler_params=pltpu.CompilerParams(use_tc_tiling_on_sc=False),
    )
    def body(p_hbm, out_hbm, p_v, sem):
        cid = lax.axis_index("c")
        sid = lax.axis_index("s")
        wid = sid * 2 + cid
        h = wid // 2
        i0 = (wid % 2) * (_S // 2)

        pltpu.sync_copy(p_hbm.at[h], p_v)

        def row_copy(i, kk):
            off = (_S - 1) - i
            q = ((_S - 1) - kk) % _NSHIFT
            base = q * _PAD + (off - q)
            return pltpu.make_async_copy(
                p_v.at[pl.ds(base, _S)],
                out_hbm.at[h, i],
                sem,
            )

        def fire(g):
            ibase = i0 + g * _K
            for kk in range(_K):
                row_copy(ibase + kk, kk).start()

        def drain(g):
            ibase = i0 + g * _K
            for kk in range(_K):
                row_copy(ibase + kk, kk).wait()

        ngroups = (_S // 2) // _K
        fire(0)
        fire(1)

        def loop(g, carry):
            drain(g)
            fire(g + 2)
            return carry

        lax.fori_loop(0, ngroups - 2, loop, 0)
        drain(ngroups - 2)
        drain(ngroups - 1)

    return body(shifted_tables)


def kernel(seq_len, bias_table):
    del seq_len
    rev = bias_table[::-1, :]
    pad_rows = _PAD + _NSHIFT - 1 - rev.shape[0]
    rev = jnp.concatenate(
        [rev, jnp.zeros((pad_rows, _H), bias_table.dtype)], axis=0)
    shifted = jnp.stack(
        [lax.slice_in_dim(rev, q, q + _PAD, axis=0) for q in range(_NSHIFT)],
        axis=0)
    shifted = jnp.transpose(shifted, (2, 0, 1)).reshape(_H, _NSHIFT * _PAD)
    out = _expand_bias(shifted)
    return out[None]
